# baseline (speedup 1.0000x reference)
import functools

import jax
import jax.numpy as jnp
from jax import lax
from jax.experimental import pallas as pl
from jax.experimental.pallas import tpu as pltpu
from jax.experimental.pallas import tpu_sc as plsc

_N = 5000
_LANES = 16
_NPAD = ((_N + _LANES - 1) // _LANES) * _LANES
_CHUNKS = _NPAD // _LANES
_IMG_W = 2048.0
_IMG_H = 2048.0
_IOU_THRESH = 0.5
_SCORE_THRESH = 0.05
_MAX_DET = 4
_MAX_COORD = max(_IMG_W, _IMG_H) + 1.0
_ROW_X1, _ROW_Y1, _ROW_X2, _ROW_Y2, _ROW_SC, _ROW_LB = range(6)


def _nms_body(pk_h, out_h,
              pkv, x1o, y1o, x2o, y2o, sv, outs):
  is_worker0 = (lax.axis_index("c") == 0) & (lax.axis_index("s") == 0)

  @pl.when(is_worker0)
  def _():
    pltpu.sync_copy(pk_h, pkv)

    lane = lax.iota(jnp.int32, _LANES)
    neg2 = jnp.full((_LANES,), -2.0, jnp.float32)
    zero_i = jnp.zeros((_LANES,), jnp.int32)

    def pre_body(base, carry):
      m, mi = carry
      bx1 = jnp.minimum(jnp.maximum(pkv[pl.ds(_ROW_X1 * _NPAD + base, _LANES)], 0.0), _IMG_W)
      by1 = jnp.minimum(jnp.maximum(pkv[pl.ds(_ROW_Y1 * _NPAD + base, _LANES)], 0.0), _IMG_H)
      bx2 = jnp.minimum(jnp.maximum(pkv[pl.ds(_ROW_X2 * _NPAD + base, _LANES)], 0.0), _IMG_W)
      by2 = jnp.minimum(jnp.maximum(pkv[pl.ds(_ROW_Y2 * _NPAD + base, _LANES)], 0.0), _IMG_H)
      lb = plsc.bitcast(pkv[pl.ds(_ROW_LB * _NPAD + base, _LANES)], jnp.int32)
      off = lb.astype(jnp.float32) * _MAX_COORD
      sl = pl.ds(base, _LANES)
      x1o[sl] = bx1 + off
      y1o[sl] = by1 + off
      x2o[sl] = bx2 + off
      y2o[sl] = by2 + off
      raw = pkv[pl.ds(_ROW_SC * _NPAD + base, _LANES)]
      s = jnp.where(raw > _SCORE_THRESH, raw, -1.0)
      sv[sl] = s
      upd = s > m
      m = jnp.where(upd, s, m)
      mi = jnp.where(upd, lane + base, mi)
      return m, mi

    m, mi = plsc.parallel_loop(
        0, _NPAD, _LANES, unroll=8, carry=(neg2, zero_i))(pre_body)

    mod4 = jnp.bitwise_and(lane, 3)
    grp4 = lax.shift_right_logical(lane, 2)
    full_box = jnp.where(mod4 <= 1, 0.0, jnp.where(mod4 == 2, _IMG_W, _IMG_H))

    ob_vec = jnp.zeros((_LANES,), jnp.float32)
    os_vec = jnp.zeros((_LANES,), jnp.float32)
    ol_vec = jnp.zeros((_LANES,), jnp.int32)

    def bcast_argmax(m, mi):
      for k in (1, 2, 4, 8):
        idx = jnp.bitwise_xor(lane, k)
        om = m.at[idx].get(mode="promise_in_bounds")
        omi = mi.at[idx].get(mode="promise_in_bounds")
        take = (om > m) | ((om == m) & (omi < mi))
        m = jnp.where(take, om, m)
        mi = jnp.where(take, omi, mi)
      return m, mi

    for d in range(_MAX_DET):
      mv, sel = bcast_argmax(m, mi)

      co_x1 = plsc.load_gather(x1o, [sel])
      co_y1 = plsc.load_gather(y1o, [sel])
      co_x2 = plsc.load_gather(x2o, [sel])
      co_y2 = plsc.load_gather(y2o, [sel])
      ca = (co_x2 - co_x1) * (co_y2 - co_y1)
      clb = plsc.bitcast(
          plsc.load_gather(pkv, [sel + _ROW_LB * _NPAD]), jnp.int32)
      coff = clb.astype(jnp.float32) * _MAX_COORD
      cx1 = co_x1 - coff
      cy1 = co_y1 - coff
      cx2 = co_x2 - coff
      cy2 = co_y2 - coff

      badv = (((cy2.astype(jnp.int32) - cy1.astype(jnp.int32)) < 1)
              | ((cx2.astype(jnp.int32) - cx1.astype(jnp.int32)) < 1)
              | (mv < 0.0))
      boxsel = jnp.where(mod4 == 0, cx1,
                         jnp.where(mod4 == 1, cy1,
                                   jnp.where(mod4 == 2, cx2, cy2)))
      boxsel = jnp.where(badv, full_box, boxsel)
      ob_vec = jnp.where(grp4 == d, boxsel, ob_vec)
      os_vec = jnp.where(lane == d, jnp.where(mv < 0.0, 0.0, mv), os_vec)
      ol_vec = jnp.where(lane == d, jnp.where(badv, 0, clb), ol_vec)

      if d + 1 < _MAX_DET:
        def sup_body(base, carry, co_x1=co_x1, co_y1=co_y1, co_x2=co_x2,
                     co_y2=co_y2, ca=ca):
          m, mi = carry
          sl = pl.ds(base, _LANES)
          xo1 = x1o[sl]
          yo1 = y1o[sl]
          xo2 = x2o[sl]
          yo2 = y2o[sl]
          ltx = jnp.maximum(co_x1, xo1)
          lty = jnp.maximum(co_y1, yo1)
          rbx = jnp.minimum(co_x2, xo2)
          rby = jnp.minimum(co_y2, yo2)
          w = jnp.maximum(rbx - ltx, 0.0)
          h = jnp.maximum(rby - lty, 0.0)
          inter = w * h
          area = (xo2 - xo1) * (yo2 - yo1)
          union = jnp.maximum(ca + area - inter, 1e-9)
          s = jnp.where(inter > _IOU_THRESH * union, -1.0, sv[sl])
          sv[sl] = s
          upd = s > m
          m = jnp.where(upd, s, m)
          mi = jnp.where(upd, lane + base, mi)
          return m, mi

        m, mi = plsc.parallel_loop(
            0, _NPAD, _LANES, unroll=8, carry=(neg2, zero_i))(sup_body)

    outs[pl.ds(0, _LANES)] = ob_vec
    outs[pl.ds(_LANES, _LANES)] = os_vec
    outs[pl.ds(2 * _LANES, _LANES)] = plsc.bitcast(ol_vec, jnp.float32)
    pltpu.sync_copy(outs, out_h)


@functools.cache
def _get_sc_kernel():
  mesh = plsc.VectorSubcoreMesh(core_axis_name="c", subcore_axis_name="s")
  f32 = jnp.float32
  return pl.kernel(
      _nms_body,
      out_type=jax.ShapeDtypeStruct((3 * _LANES,), f32),
      mesh=mesh,
      compiler_params=pltpu.CompilerParams(needs_layout_passes=False),
      scratch_types=[
          pltpu.VMEM((6 * _NPAD,), f32),
          pltpu.VMEM((_NPAD,), f32),
          pltpu.VMEM((_NPAD,), f32),
          pltpu.VMEM((_NPAD,), f32),
          pltpu.VMEM((_NPAD,), f32),
          pltpu.VMEM((_NPAD,), f32),
          pltpu.VMEM((3 * _LANES,), f32),
      ],
  )


def kernel(boxes, scores, labels):
  pad = _NPAD - boxes.shape[0]
  cols = jnp.pad(boxes, ((0, pad), (0, 0))).T.reshape(-1)
  sc = jnp.pad(scores, (0, pad))
  lbf = lax.bitcast_convert_type(jnp.pad(labels, (0, pad)), jnp.float32)
  packed = jnp.concatenate([cols, sc, lbf])
  out = _get_sc_kernel()(packed)
  ob = out[: _LANES].reshape(_MAX_DET, 4)
  osc = out[_LANES : _LANES + _MAX_DET]
  olb = lax.bitcast_convert_type(
      out[2 * _LANES : 2 * _LANES + _MAX_DET], jnp.int32)
  return (ob, osc, olb)


# speedup vs baseline: 1.2141x; 1.2141x over previous
import functools

import jax
import jax.numpy as jnp
from jax import lax
from jax.experimental import pallas as pl
from jax.experimental.pallas import tpu as pltpu
from jax.experimental.pallas import tpu_sc as plsc

_N = 5000
_LANES = 16
_NTILES = 16
_NPAD = 5120
_PER = _NPAD // _NTILES
_IMG_W = 2048.0
_IMG_H = 2048.0
_IOU_THRESH = 0.5
_SCORE_THRESH = 0.05
_MAX_DET = 4
_MAX_COORD = max(_IMG_W, _IMG_H) + 1.0
_ROW_X1, _ROW_Y1, _ROW_X2, _ROW_Y2, _ROW_SC, _ROW_LB = range(6)
_NFLD = 8


def _nms_body(pk_h, out_h,
              pkv, x1o, y1o, x2o, y2o, sv, recv, rec_all, outs, sh_rec, sem):
  wid = lax.axis_index("s")

  @pl.when(lax.axis_index("c") == 0)
  def _():
    copies = [
        pltpu.make_async_copy(
            pk_h.at[pl.ds(r * _NPAD + wid * _PER, _PER)],
            pkv.at[pl.ds(r * _PER, _PER)],
            sem,
        )
        for r in range(6)
    ]
    for c in copies:
      c.start()
    for c in copies:
      c.wait()

    lane = lax.iota(jnp.int32, _LANES)
    neg2 = jnp.full((_LANES,), -2.0, jnp.float32)
    zero_i = jnp.zeros((_LANES,), jnp.int32)

    def pre_body(base, carry):
      m, mi = carry
      bx1 = jnp.minimum(jnp.maximum(pkv[pl.ds(_ROW_X1 * _PER + base, _LANES)], 0.0), _IMG_W)
      by1 = jnp.minimum(jnp.maximum(pkv[pl.ds(_ROW_Y1 * _PER + base, _LANES)], 0.0), _IMG_H)
      bx2 = jnp.minimum(jnp.maximum(pkv[pl.ds(_ROW_X2 * _PER + base, _LANES)], 0.0), _IMG_W)
      by2 = jnp.minimum(jnp.maximum(pkv[pl.ds(_ROW_Y2 * _PER + base, _LANES)], 0.0), _IMG_H)
      lb = plsc.bitcast(pkv[pl.ds(_ROW_LB * _PER + base, _LANES)], jnp.int32)
      off = lb.astype(jnp.float32) * _MAX_COORD
      sl = pl.ds(base, _LANES)
      x1o[sl] = bx1 + off
      y1o[sl] = by1 + off
      x2o[sl] = bx2 + off
      y2o[sl] = by2 + off
      raw = pkv[pl.ds(_ROW_SC * _PER + base, _LANES)]
      s = jnp.where(raw > _SCORE_THRESH, raw, -1.0)
      sv[sl] = s
      upd = s > m
      m = jnp.where(upd, s, m)
      mi = jnp.where(upd, lane + base, mi)
      return m, mi

    m, mi = plsc.parallel_loop(
        0, _PER, _LANES, unroll=4, carry=(neg2, zero_i))(pre_body)

    mod4 = jnp.bitwise_and(lane, 3)
    grp4 = lax.shift_right_logical(lane, 2)
    full_box = jnp.where(mod4 <= 1, 0.0, jnp.where(mod4 == 2, _IMG_W, _IMG_H))

    ob_vec = jnp.zeros((_LANES,), jnp.float32)
    os_vec = jnp.zeros((_LANES,), jnp.float32)
    ol_vec = jnp.zeros((_LANES,), jnp.int32)

    def gat(vec, idx):
      return vec.at[idx].get(mode="promise_in_bounds")

    def local_argmax(m, mi):
      for k in (1, 2, 4, 8):
        idx = jnp.bitwise_xor(lane, k)
        om = gat(m, idx)
        omi = gat(mi, idx)
        take = (om > m) | ((om == m) & (omi < mi))
        m = jnp.where(take, om, m)
        mi = jnp.where(take, omi, mi)
      return m, mi

    for d in range(_MAX_DET):
      lm, lli = local_argmax(m, mi)

      gm = lm
      gmi = (lli + wid * _PER).astype(jnp.float32)
      lx1 = plsc.load_gather(x1o, [lli])
      ly1 = plsc.load_gather(y1o, [lli])
      lx2 = plsc.load_gather(x2o, [lli])
      ly2 = plsc.load_gather(y2o, [lli])
      llb = plsc.bitcast(
          plsc.load_gather(pkv, [lli + _ROW_LB * _PER]),
          jnp.int32).astype(jnp.float32)
      rec = jnp.where(lane == 0, gm,
            jnp.where(lane == 1, gmi,
            jnp.where(lane == 2, lx1,
            jnp.where(lane == 3, ly1,
            jnp.where(lane == 4, lx2,
            jnp.where(lane == 5, ly2,
            jnp.where(lane == 6, llb, 0.0)))))))
      recv[...] = rec
      pltpu.sync_copy(recv.at[pl.ds(0, _NFLD)],
                      sh_rec.at[pl.ds(wid * _NFLD, _NFLD)])
      plsc.subcore_barrier()
      pltpu.sync_copy(sh_rec, rec_all)
      plsc.subcore_barrier()

      wm = plsc.load_gather(rec_all, [lane * _NFLD + 0])
      wmi = plsc.load_gather(rec_all, [lane * _NFLD + 1])
      wx1 = plsc.load_gather(rec_all, [lane * _NFLD + 2])
      wy1 = plsc.load_gather(rec_all, [lane * _NFLD + 3])
      wx2 = plsc.load_gather(rec_all, [lane * _NFLD + 4])
      wy2 = plsc.load_gather(rec_all, [lane * _NFLD + 5])
      wlb = plsc.load_gather(rec_all, [lane * _NFLD + 6])
      for k in (1, 2, 4, 8):
        idx = jnp.bitwise_xor(lane, k)
        om = gat(wm, idx)
        omi = gat(wmi, idx)
        take = (om > wm) | ((om == wm) & (omi < wmi))
        wm = jnp.where(take, om, wm)
        wmi = jnp.where(take, omi, wmi)
        wx1 = jnp.where(take, gat(wx1, idx), wx1)
        wy1 = jnp.where(take, gat(wy1, idx), wy1)
        wx2 = jnp.where(take, gat(wx2, idx), wx2)
        wy2 = jnp.where(take, gat(wy2, idx), wy2)
        wlb = jnp.where(take, gat(wlb, idx), wlb)

      ca = (wx2 - wx1) * (wy2 - wy1)
      clbi = wlb.astype(jnp.int32)
      coff = wlb * _MAX_COORD
      cx1 = wx1 - coff
      cy1 = wy1 - coff
      cx2 = wx2 - coff
      cy2 = wy2 - coff

      badv = (((cy2.astype(jnp.int32) - cy1.astype(jnp.int32)) < 1)
              | ((cx2.astype(jnp.int32) - cx1.astype(jnp.int32)) < 1)
              | (wm < 0.0))
      boxsel = jnp.where(mod4 == 0, cx1,
                         jnp.where(mod4 == 1, cy1,
                                   jnp.where(mod4 == 2, cx2, cy2)))
      boxsel = jnp.where(badv, full_box, boxsel)
      ob_vec = jnp.where(grp4 == d, boxsel, ob_vec)
      os_vec = jnp.where(lane == d, jnp.where(wm < 0.0, 0.0, wm), os_vec)
      ol_vec = jnp.where(lane == d, jnp.where(badv, 0, clbi), ol_vec)

      if d + 1 < _MAX_DET:
        def sup_body(base, carry, wx1=wx1, wy1=wy1, wx2=wx2, wy2=wy2, ca=ca):
          m, mi = carry
          sl = pl.ds(base, _LANES)
          xo1 = x1o[sl]
          yo1 = y1o[sl]
          xo2 = x2o[sl]
          yo2 = y2o[sl]
          ltx = jnp.maximum(wx1, xo1)
          lty = jnp.maximum(wy1, yo1)
          rbx = jnp.minimum(wx2, xo2)
          rby = jnp.minimum(wy2, yo2)
          w = jnp.maximum(rbx - ltx, 0.0)
          h = jnp.maximum(rby - lty, 0.0)
          inter = w * h
          area = (xo2 - xo1) * (yo2 - yo1)
          union = jnp.maximum(ca + area - inter, 1e-9)
          s = jnp.where(inter > _IOU_THRESH * union, -1.0, sv[sl])
          sv[sl] = s
          upd = s > m
          m = jnp.where(upd, s, m)
          mi = jnp.where(upd, lane + base, mi)
          return m, mi

        m, mi = plsc.parallel_loop(
            0, _PER, _LANES, unroll=4, carry=(neg2, zero_i))(sup_body)

    @pl.when(wid == 0)
    def _():
      outs[pl.ds(0, _LANES)] = ob_vec
      outs[pl.ds(_LANES, _LANES)] = os_vec
      outs[pl.ds(2 * _LANES, _LANES)] = plsc.bitcast(ol_vec, jnp.float32)
      pltpu.sync_copy(outs, out_h)


@functools.cache
def _get_sc_kernel():
  mesh = plsc.VectorSubcoreMesh(core_axis_name="c", subcore_axis_name="s")
  f32 = jnp.float32
  return pl.kernel(
      _nms_body,
      out_type=jax.ShapeDtypeStruct((3 * _LANES,), f32),
      mesh=mesh,
      compiler_params=pltpu.CompilerParams(needs_layout_passes=False),
      scratch_types=[
          pltpu.VMEM((6 * _PER,), f32),
          pltpu.VMEM((_PER,), f32),
          pltpu.VMEM((_PER,), f32),
          pltpu.VMEM((_PER,), f32),
          pltpu.VMEM((_PER,), f32),
          pltpu.VMEM((_PER,), f32),
          pltpu.VMEM((_LANES,), f32),
          pltpu.VMEM((_NTILES * _NFLD,), f32),
          pltpu.VMEM((3 * _LANES,), f32),
          pltpu.VMEM_SHARED((_NTILES * _NFLD,), f32),
          pltpu.SemaphoreType.DMA,
      ],
  )


def kernel(boxes, scores, labels):
  pad = _NPAD - boxes.shape[0]
  cols = jnp.pad(boxes, ((0, pad), (0, 0))).T.reshape(-1)
  sc = jnp.pad(scores, (0, pad))
  lbf = lax.bitcast_convert_type(jnp.pad(labels, (0, pad)), jnp.float32)
  packed = jnp.concatenate([cols, sc, lbf])
  out = _get_sc_kernel()(packed)
  ob = out[: _LANES].reshape(_MAX_DET, 4)
  osc = out[_LANES : _LANES + _MAX_DET]
  olb = lax.bitcast_convert_type(
      out[2 * _LANES : 2 * _LANES + _MAX_DET], jnp.int32)
  return (ob, osc, olb)
